# Initial kernel scaffold; baseline (speedup 1.0000x reference)
import jax
import jax.numpy as jnp
from jax.experimental import pallas as pl


def kernel(hidden_states, w1, w2, topk_weights, topk_ids):
    raise NotImplementedError("write your pallas kernel here")


# speedup vs baseline: 1.2454x; 1.2454x over previous
import jax
import jax.numpy as jnp
from jax.experimental import pallas as pl
from jax.experimental.pallas import tpu as pltpu


def _moe_body(x_ref, w1_ref, w2_ref, tw_ref, ids_ref, out_ref):
    e = pl.program_id(0)
    n = w2_ref.shape[2]
    x = x_ref[...]
    h = jax.lax.dot_general(
        x, w1_ref[0], (((1,), (1,)), ((), ())),
        preferred_element_type=jnp.float32)
    gate = h[:, :n]
    up = h[:, n:]
    act = gate * jax.nn.sigmoid(gate) * up
    y = jax.lax.dot_general(
        act, w2_ref[0], (((1,), (1,)), ((), ())),
        preferred_element_type=jnp.float32)
    sel = (ids_ref[...] == e).astype(jnp.float32)
    wpe = jnp.sum(tw_ref[...] * sel, axis=1, keepdims=True)
    contrib = wpe * y

    @pl.when(e == 0)
    def _init():
        out_ref[...] = contrib

    @pl.when(e > 0)
    def _acc():
        out_ref[...] += contrib


def kernel(hidden_states, w1, w2, topk_weights, topk_ids):
    m, k = hidden_states.shape
    e_total, two_n, _ = w1.shape
    n = w2.shape[2]
    topk = topk_ids.shape[1]
    return pl.pallas_call(
        _moe_body,
        grid=(e_total,),
        in_specs=[
            pl.BlockSpec((m, k), lambda e: (0, 0)),
            pl.BlockSpec((1, two_n, k), lambda e: (e, 0, 0)),
            pl.BlockSpec((1, k, n), lambda e: (e, 0, 0)),
            pl.BlockSpec((m, topk), lambda e: (0, 0)),
            pl.BlockSpec((m, topk), lambda e: (0, 0)),
        ],
        out_specs=pl.BlockSpec((m, k), lambda e: (0, 0)),
        out_shape=jax.ShapeDtypeStruct((m, k), jnp.float32),
        compiler_params=pltpu.CompilerParams(
            dimension_semantics=("arbitrary",)),
    )(hidden_states, w1, w2, topk_weights, topk_ids)
